# baseline (speedup 1.0000x reference)
import jax
import jax.numpy as jnp
from jax.experimental import pallas as pl
from jax.experimental.pallas import tpu as pltpu

_E = 8
_K = 2


def _router_kernel(x_ref, w_ref, rw_ref, se_ref, cnt_ref, ps_ref):
    x = x_ref[...]
    w = w_ref[...]
    logits = jax.lax.dot_general(
        w, x, (((1,), (1,)), ((), ())),
        preferred_element_type=jnp.float32)

    eidx = jax.lax.broadcasted_iota(jnp.int32, logits.shape, 0)
    m1 = jnp.max(logits, axis=0, keepdims=True)
    i1 = jnp.min(jnp.where(logits == m1, eidx, _E), axis=0, keepdims=True)
    masked = jnp.where(eidx == i1, -jnp.inf, logits)
    m2 = jnp.max(masked, axis=0, keepdims=True)
    i2 = jnp.min(jnp.where(masked == m2, eidx, _E), axis=0, keepdims=True)

    e2 = jnp.exp(m2 - m1)
    denom = 1.0 + e2
    rw_ref[...] = jnp.concatenate([1.0 / denom, e2 / denom], axis=0)
    se_ref[...] = jnp.concatenate([i1, i2], axis=0)

    ex = jnp.exp(logits - m1)
    probs = ex / jnp.sum(ex, axis=0, keepdims=True)
    hit = ((eidx == i1) | (eidx == i2)).astype(jnp.float32)
    cnt_ref[...] = jnp.sum(hit, axis=1, keepdims=True)[None]
    ps_ref[...] = jnp.sum(probs, axis=1, keepdims=True)[None]


def _aux_kernel(n_tokens, cnt_ref, ps_ref, aux_ref):
    f = jnp.sum(cnt_ref[...], axis=0) / (n_tokens * _K)
    p_mean = jnp.sum(ps_ref[...], axis=0) / n_tokens
    aux_ref[...] = jnp.reshape(_E * jnp.sum(f * p_mean), (1, 1))


def kernel(hidden_states, W):
    import functools
    n, d = hidden_states.shape
    block = 2048
    nb = n // block

    rw, se, cnt, ps = pl.pallas_call(
        _router_kernel,
        grid=(nb,),
        in_specs=[
            pl.BlockSpec((block, d), lambda i: (i, 0)),
            pl.BlockSpec((_E, d), lambda i: (0, 0)),
        ],
        out_specs=[
            pl.BlockSpec((_K, block), lambda i: (0, i)),
            pl.BlockSpec((_K, block), lambda i: (0, i)),
            pl.BlockSpec((1, _E, 1), lambda i: (i, 0, 0)),
            pl.BlockSpec((1, _E, 1), lambda i: (i, 0, 0)),
        ],
        out_shape=[
            jax.ShapeDtypeStruct((_K, n), jnp.float32),
            jax.ShapeDtypeStruct((_K, n), jnp.int32),
            jax.ShapeDtypeStruct((nb, _E, 1), jnp.float32),
            jax.ShapeDtypeStruct((nb, _E, 1), jnp.float32),
        ],
        compiler_params=pltpu.CompilerParams(
            dimension_semantics=("parallel",),
        ),
    )(hidden_states, W)

    aux = pl.pallas_call(
        functools.partial(_aux_kernel, n),
        out_shape=jax.ShapeDtypeStruct((1, 1), jnp.float32),
    )(cnt, ps)
    return (rw.T, se.T, aux.reshape(()))
